# baseline (speedup 1.0000x reference)
import functools

import jax
import jax.numpy as jnp
from jax import lax
from jax.experimental import pallas as pl
from jax.experimental.pallas import tpu as pltpu
from jax.experimental.pallas import tpu_sc as plsc

D_MODEL = 64
D_VOCAB = 1000000
D_PAD = 128


_TBLK = 512


def _transpose_body(w_ref, out_ref):
    w = w_ref[...]
    r = lax.broadcasted_iota(jnp.int32, (D_MODEL, D_MODEL), 0)
    c = lax.broadcasted_iota(jnp.int32, (D_MODEL, D_MODEL), 1)
    eye = (r == c).astype(jnp.float32)
    t = lax.dot_general(w, eye, (((0,), (0,)), ((), ())),
                        precision=lax.Precision.HIGHEST,
                        preferred_element_type=jnp.float32)
    out_ref[...] = jnp.concatenate(
        [t, jnp.zeros((_TBLK, D_PAD - D_MODEL), jnp.float32)], axis=1)


def _transpose_table(W_E):
    return pl.pallas_call(
        _transpose_body,
        grid=(pl.cdiv(D_VOCAB, _TBLK),),
        in_specs=[pl.BlockSpec((D_MODEL, _TBLK), lambda i: (0, i))],
        out_specs=pl.BlockSpec((_TBLK, D_PAD), lambda i: (i, 0)),
        out_shape=jax.ShapeDtypeStruct((D_VOCAB, D_PAD), jnp.float32),
    )(W_E)


_CHUNK = 512


def _make_gather(B):
    info = plsc.get_sparse_core_info()
    NW = info.num_cores * info.num_subcores
    b_per_w = B // NW
    n_chunks = b_per_w // _CHUNK
    mesh = plsc.VectorSubcoreMesh(core_axis_name="c", subcore_axis_name="s")

    @functools.partial(
        pl.kernel,
        mesh=mesh,
        compiler_params=pltpu.CompilerParams(use_tc_tiling_on_sc=True),
        out_type=jax.ShapeDtypeStruct((B, D_PAD), jnp.float32),
        scratch_types=[
            pltpu.VMEM((_CHUNK,), jnp.int32),
            pltpu.VMEM((_CHUNK, D_PAD), jnp.float32),
            pltpu.SemaphoreType.DMA,
        ],
    )
    def gather_kernel(table_hbm, idx_hbm, out_hbm, idx_v, rows_v, sem):
        wid = lax.axis_index("s") * info.num_cores + lax.axis_index("c")
        wbase = wid * b_per_w

        def body(c, carry):
            base = wbase + c * _CHUNK
            pltpu.sync_copy(idx_hbm.at[pl.ds(base, _CHUNK)], idx_v)
            pltpu.async_copy(table_hbm.at[idx_v], rows_v, sem).wait()
            pltpu.sync_copy(rows_v, out_hbm.at[pl.ds(base, _CHUNK)])
            return carry

        lax.fori_loop(0, n_chunks, body, 0)

    return gather_kernel


def kernel(x, W_E):
    b, p = x.shape
    W_T = _transpose_table(W_E)
    idx = x.reshape(-1).astype(jnp.int32)
    out = _make_gather(b * p)(W_T, idx)
    return out[:, :D_MODEL].reshape(b, p, D_MODEL)


# speedup vs baseline: 1.8406x; 1.8406x over previous
import functools

import jax
import jax.numpy as jnp
from jax import lax
from jax.experimental import pallas as pl
from jax.experimental.pallas import tpu as pltpu
from jax.experimental.pallas import tpu_sc as plsc

D_MODEL = 64
D_VOCAB = 1000000
D_PAD = 128


_TBLK = 512


def _transpose_body(w_ref, out_ref):
    w = w_ref[...]
    r = lax.broadcasted_iota(jnp.int32, (D_MODEL, D_MODEL), 0)
    c = lax.broadcasted_iota(jnp.int32, (D_MODEL, D_MODEL), 1)
    eye = (r == c).astype(jnp.float32)
    t = lax.dot_general(w, eye, (((0,), (0,)), ((), ())),
                        precision=lax.Precision.HIGHEST,
                        preferred_element_type=jnp.float32)
    out_ref[...] = jnp.concatenate(
        [t, jnp.zeros((_TBLK, D_PAD - D_MODEL), jnp.float32)], axis=1)


def _transpose_table(W_E):
    return pl.pallas_call(
        _transpose_body,
        grid=(pl.cdiv(D_VOCAB, _TBLK),),
        in_specs=[pl.BlockSpec((D_MODEL, _TBLK), lambda i: (0, i))],
        out_specs=pl.BlockSpec((_TBLK, D_PAD), lambda i: (i, 0)),
        out_shape=jax.ShapeDtypeStruct((D_VOCAB, D_PAD), jnp.float32),
    )(W_E)


_CHUNK = 512


def _make_gather(B):
    info = plsc.get_sparse_core_info()
    NW = info.num_cores * info.num_subcores
    b_per_w = B // NW
    n_chunks = b_per_w // _CHUNK
    mesh = plsc.VectorSubcoreMesh(core_axis_name="c", subcore_axis_name="s")

    @functools.partial(
        pl.kernel,
        mesh=mesh,
        compiler_params=pltpu.CompilerParams(use_tc_tiling_on_sc=True),
        out_type=jax.ShapeDtypeStruct((B, D_PAD), jnp.float32),
        scratch_types=[
            pltpu.VMEM((_CHUNK,), jnp.int32),
            pltpu.VMEM((_CHUNK, D_PAD), jnp.float32),
            pltpu.SemaphoreType.DMA,
        ],
    )
    def gather_kernel(table_hbm, idx_hbm, out_hbm, idx_v, rows_v, sem):
        wid = lax.axis_index("s") * info.num_cores + lax.axis_index("c")
        wbase = wid * b_per_w

        def body(c, carry):
            base = wbase + c * _CHUNK
            pltpu.sync_copy(idx_hbm.at[pl.ds(base, _CHUNK)], idx_v)
            pltpu.async_copy(table_hbm.at[idx_v], rows_v, sem).wait()
            pltpu.sync_copy(rows_v, out_hbm.at[pl.ds(base, _CHUNK)])
            return carry

        lax.fori_loop(0, n_chunks, body, 0)

    return gather_kernel


def kernel(x, W_E):
    b, p = x.shape
    W_T = jnp.pad(jnp.swapaxes(W_E, 0, 1), ((0, 0), (0, D_PAD - D_MODEL)))
    idx = x.reshape(-1).astype(jnp.int32)
    out = _make_gather(b * p)(W_T, idx)
    return out[:, :D_MODEL].reshape(b, p, D_MODEL)
